# baseline (speedup 1.0000x reference)
import functools

import jax
import jax.numpy as jnp
from jax import lax
from jax.experimental import pallas as pl
from jax.experimental.pallas import tpu as pltpu
from jax.experimental.pallas import tpu_sc as plsc

B = 4096
L = 200
D = 32
VOCAB = 1000000
NC = 2
NS = 16
NW = NC * NS
SPW = B // NW
CHUNK = 100
NCH = L // CHUNK
NBUF = 2


def _pool_body(idx_hbm, off_hbm, table_hbm, out_hbm, idx_v, off_v, rows0,
               rows1, out_v, sem0, sem1):
    wid = lax.axis_index("s") * NC + lax.axis_index("c")

    pltpu.sync_copy(idx_hbm.at[pl.ds(wid * SPW * NCH, SPW * NCH)], idx_v)
    pltpu.sync_copy(off_hbm.at[pl.ds(wid * SPW * L, SPW * L)], off_v)

    rows = (rows0, rows1)
    sems = (sem0, sem1)

    def issue(s, b):
        for c in range(NCH):
            pltpu.async_copy(
                table_hbm.at[idx_v.at[s * NCH + c]],
                rows[b].at[pl.ds(c * CHUNK, CHUNK)],
                sems[b],
            )

    def wait(b):
        for c in range(NCH):
            pltpu.make_async_copy(
                table_hbm.at[idx_v.at[c]],
                rows[b].at[pl.ds(c * CHUNK, CHUNK)],
                sems[b],
            ).wait()

    for b in range(NBUF):
        issue(b, b)

    zeros = jnp.zeros((16,), jnp.float32)
    neginf = jnp.full((16,), -jnp.inf, jnp.float32)
    inv_l = jnp.float32(1.0 / L)
    iota = lax.iota(jnp.int32, 16)

    def outer(i, _):
        for b in range(NBUF):
            s = NBUF * i + b
            rb = rows[b]
            wait(b)
            tok0 = s * L

            @plsc.parallel_loop(
                0, L, step=2, unroll=2,
                carry=(zeros, zeros, neginf, neginf,
                       zeros, zeros, neginf, neginf))
            def red(r, carry):
                sa0, sa1, ma0, ma1, sb0, sb1, mb0, mb1 = carry
                offa = plsc.load_gather(
                    off_v, [jnp.full((16,), tok0 + r, jnp.int32)])
                offb = plsc.load_gather(
                    off_v, [jnp.full((16,), tok0 + r + 1, jnp.int32)])
                ra = jnp.full((16,), r, jnp.int32)
                rbv = ra + 1
                ca = offa + iota
                cb = offb + iota
                va0 = plsc.load_gather(rb, [ra, ca])
                va1 = plsc.load_gather(rb, [ra, ca + 16])
                vb0 = plsc.load_gather(rb, [rbv, cb])
                vb1 = plsc.load_gather(rb, [rbv, cb + 16])
                return (sa0 + va0, sa1 + va1,
                        jnp.maximum(ma0, va0), jnp.maximum(ma1, va1),
                        sb0 + vb0, sb1 + vb1,
                        jnp.maximum(mb0, vb0), jnp.maximum(mb1, vb1))

            sa0, sa1, ma0, ma1, sb0, sb1, mb0, mb1 = red

            @pl.when(s + NBUF < SPW)
            def _():
                issue(s + NBUF, b)

            out_v[s, pl.ds(0, 16)] = (sa0 + sb0) * inv_l
            out_v[s, pl.ds(16, 16)] = (sa1 + sb1) * inv_l
            out_v[s, pl.ds(32, 16)] = jnp.maximum(ma0, mb0)
            out_v[s, pl.ds(48, 16)] = jnp.maximum(ma1, mb1)
        return 0

    lax.fori_loop(0, SPW // NBUF, outer, 0)

    pltpu.sync_copy(out_v, out_hbm.at[pl.ds(wid * SPW, SPW)])


def _pooled_sc(idx4, off32, table128):
    mesh = plsc.VectorSubcoreMesh(core_axis_name="c", subcore_axis_name="s")
    f = pl.kernel(
        _pool_body,
        out_type=jax.ShapeDtypeStruct((B, 2 * D), jnp.float32),
        mesh=mesh,
        scratch_types=[
            pltpu.VMEM((SPW * NCH, CHUNK), jnp.int32),
            pltpu.VMEM((SPW * L,), jnp.int32),
            pltpu.VMEM((L, 4 * D), jnp.float32),
            pltpu.VMEM((L, 4 * D), jnp.float32),
            pltpu.VMEM((SPW, 2 * D), jnp.float32),
            pltpu.SemaphoreType.DMA,
            pltpu.SemaphoreType.DMA,
        ],
        compiler_params=pltpu.CompilerParams(
            use_tc_tiling_on_sc=False, needs_layout_passes=False),
    )
    return f(idx4, off32, table128)


def _mlp_body(p_ref, w1_ref, b1_ref, w2_ref, b2_ref, w3_ref, b3_ref, o_ref):
    h = jnp.dot(p_ref[...], w1_ref[...], preferred_element_type=jnp.float32)
    h = jnp.maximum(h + b1_ref[...], 0.0)
    h = jnp.dot(h, w2_ref[...], preferred_element_type=jnp.float32)
    h = jnp.maximum(h + b2_ref[...], 0.0)
    h = jnp.dot(h, w3_ref[...], preferred_element_type=jnp.float32)
    o_ref[...] = jax.nn.sigmoid(h + b3_ref[...])


def _mlp_tc(pooled, W1, b1, W2, b2, W3, b3):
    blk = 1024
    return pl.pallas_call(
        _mlp_body,
        grid=(B // blk,),
        in_specs=[
            pl.BlockSpec((blk, 2 * D), lambda i: (i, 0)),
            pl.BlockSpec((2 * D, 128), lambda i: (0, 0)),
            pl.BlockSpec((1, 128), lambda i: (0, 0)),
            pl.BlockSpec((128, 64), lambda i: (0, 0)),
            pl.BlockSpec((1, 64), lambda i: (0, 0)),
            pl.BlockSpec((64, 1), lambda i: (0, 0)),
            pl.BlockSpec((1, 1), lambda i: (0, 0)),
        ],
        out_specs=pl.BlockSpec((blk, 1), lambda i: (i, 0)),
        out_shape=jax.ShapeDtypeStruct((B, 1), jnp.float32),
    )(pooled, W1, b1.reshape(1, 128), W2, b2.reshape(1, 64),
      W3, b3.reshape(1, 1))


def kernel(x, table, W1, b1, W2, b2, W3, b3):
    xi = x.astype(jnp.int32)
    idx4 = (xi // 4).reshape(B * NCH, CHUNK)
    off32 = ((xi % 4) * D).reshape(B * L)
    table128 = table.reshape(VOCAB // 4, 4 * D)
    pooled = _pooled_sc(idx4, off32, table128)
    return _mlp_tc(pooled, W1, b1, W2, b2, W3, b3)


# speedup vs baseline: 1.8181x; 1.8181x over previous
import functools

import jax
import jax.numpy as jnp
from jax import lax
from jax.experimental import pallas as pl
from jax.experimental.pallas import tpu as pltpu
from jax.experimental.pallas import tpu_sc as plsc

B = 4096
L = 200
D = 32
VOCAB = 1000000
NC = 2
NS = 16
NW = NC * NS
SPW = B // NW
CHUNK = 100
NCH = L // CHUNK
NBUF = 4
VBLK = 8192
VSUB = 512


def _relayout_body(t_ref, o_ref):
    x = t_ref[...]
    rr = VSUB // 4
    iv = lax.broadcasted_iota(jnp.int32, (VSUB, rr), 0)
    ir = lax.broadcasted_iota(jnp.int32, (VSUB, rr), 1)
    qs = [(iv == 4 * ir + g).astype(jnp.float32) for g in range(4)]
    for c in range(VBLK // VSUB):
        xc = x[:, c * VSUB:(c + 1) * VSUB]
        for g in range(4):
            z = jnp.dot(xc, qs[g], preferred_element_type=jnp.float32)
            o_ref[pl.ds(c * rr, rr), pl.ds(g * D, D)] = z.T


def _table_to_lin(table_t):
    return pl.pallas_call(
        _relayout_body,
        grid=((VOCAB + VBLK - 1) // VBLK,),
        in_specs=[pl.BlockSpec((D, VBLK), lambda i: (0, i))],
        out_specs=pl.BlockSpec((VBLK // 4, 4 * D), lambda i: (i, 0)),
        out_shape=jax.ShapeDtypeStruct((VOCAB // 4, 4 * D), jnp.float32),
    )(table_t)


def _pool_body(x_hbm, table_hbm, out_hbm, idx_v, rows0, rows1, rows2, rows3,
               out_v, sem0, sem1, sem2, sem3):
    wid = lax.axis_index("s") * NC + lax.axis_index("c")
    base_row = wid * SPW * NCH

    pltpu.sync_copy(x_hbm.at[pl.ds(base_row, SPW * NCH)], idx_v)

    rows = (rows0, rows1, rows2, rows3)
    sems = (sem0, sem1, sem2, sem3)

    def issue(s, b):
        for c in range(NCH):
            pltpu.async_copy(
                table_hbm.at[idx_v.at[s * NCH + c]],
                rows[b].at[pl.ds(c * CHUNK, CHUNK)],
                sems[b],
            )

    def wait(b):
        for c in range(NCH):
            pltpu.make_async_copy(
                table_hbm.at[idx_v.at[c]],
                rows[b].at[pl.ds(c * CHUNK, CHUNK)],
                sems[b],
            ).wait()

    for b in range(NBUF):
        issue(b, b)

    zeros = jnp.zeros((16,), jnp.float32)
    neginf = jnp.full((16,), -jnp.inf, jnp.float32)
    inv_l = jnp.float32(1.0 / L)

    def outer(i, _):
        for b in range(NBUF):
            s = NBUF * i + b
            rb = rows[b]
            wait(b)

            @plsc.parallel_loop(
                0, L, step=2, unroll=4,
                carry=(zeros, zeros, neginf, neginf,
                       zeros, zeros, neginf, neginf))
            def red(r, carry):
                sa0, sa1, ma0, ma1, sb0, sb1, mb0, mb1 = carry
                va0 = rb[r, pl.ds(0, 16)]
                va1 = rb[r, pl.ds(16, 16)]
                vb0 = rb[r + 1, pl.ds(0, 16)]
                vb1 = rb[r + 1, pl.ds(16, 16)]
                return (sa0 + va0, sa1 + va1,
                        jnp.maximum(ma0, va0), jnp.maximum(ma1, va1),
                        sb0 + vb0, sb1 + vb1,
                        jnp.maximum(mb0, vb0), jnp.maximum(mb1, vb1))

            sa0, sa1, ma0, ma1, sb0, sb1, mb0, mb1 = red

            @pl.when(s + NBUF < SPW)
            def _():
                issue(s + NBUF, b)

            out_v[s, pl.ds(0, 16)] = (sa0 + sb0) * inv_l
            out_v[s, pl.ds(16, 16)] = (sa1 + sb1) * inv_l
            out_v[s, pl.ds(32, 16)] = jnp.maximum(ma0, mb0)
            out_v[s, pl.ds(48, 16)] = jnp.maximum(ma1, mb1)
        return 0

    lax.fori_loop(0, SPW // NBUF, outer, 0)

    pltpu.sync_copy(out_v, out_hbm.at[pl.ds(wid * SPW, SPW)])


def _pooled_sc(x_idx, table_lin):
    mesh = plsc.VectorSubcoreMesh(core_axis_name="c", subcore_axis_name="s")
    f = pl.kernel(
        _pool_body,
        out_type=jax.ShapeDtypeStruct((B, 2 * D), jnp.float32),
        mesh=mesh,
        scratch_types=[
            pltpu.VMEM((SPW * NCH, CHUNK), jnp.int32),
            pltpu.VMEM((L, D), jnp.float32),
            pltpu.VMEM((L, D), jnp.float32),
            pltpu.VMEM((L, D), jnp.float32),
            pltpu.VMEM((L, D), jnp.float32),
            pltpu.VMEM((SPW, 2 * D), jnp.float32),
            pltpu.SemaphoreType.DMA,
            pltpu.SemaphoreType.DMA,
            pltpu.SemaphoreType.DMA,
            pltpu.SemaphoreType.DMA,
        ],
        compiler_params=pltpu.CompilerParams(use_tc_tiling_on_sc=False),
    )
    return f(x_idx, table_lin)


def _mlp_body(p_ref, w1_ref, b1_ref, w2_ref, b2_ref, w3_ref, b3_ref, o_ref):
    h = jnp.dot(p_ref[...], w1_ref[...], preferred_element_type=jnp.float32)
    h = jnp.maximum(h + b1_ref[...], 0.0)
    h = jnp.dot(h, w2_ref[...], preferred_element_type=jnp.float32)
    h = jnp.maximum(h + b2_ref[...], 0.0)
    h = jnp.dot(h, w3_ref[...], preferred_element_type=jnp.float32)
    o_ref[...] = jax.nn.sigmoid(h + b3_ref[...])


def _mlp_tc(pooled, W1, b1, W2, b2, W3, b3):
    blk = 1024
    return pl.pallas_call(
        _mlp_body,
        grid=(B // blk,),
        in_specs=[
            pl.BlockSpec((blk, 2 * D), lambda i: (i, 0)),
            pl.BlockSpec((2 * D, 128), lambda i: (0, 0)),
            pl.BlockSpec((1, 128), lambda i: (0, 0)),
            pl.BlockSpec((128, 64), lambda i: (0, 0)),
            pl.BlockSpec((1, 64), lambda i: (0, 0)),
            pl.BlockSpec((64, 1), lambda i: (0, 0)),
            pl.BlockSpec((1, 1), lambda i: (0, 0)),
        ],
        out_specs=pl.BlockSpec((blk, 1), lambda i: (i, 0)),
        out_shape=jax.ShapeDtypeStruct((B, 1), jnp.float32),
    )(pooled, W1, b1.reshape(1, 128), W2, b2.reshape(1, 64),
      W3, b3.reshape(1, 1))


def kernel(x, table, W1, b1, W2, b2, W3, b3):
    x_idx = x.astype(jnp.int32).reshape(B * NCH, CHUNK)
    table_lin = _table_to_lin(table.T).reshape(VOCAB, D)
    pooled = _pooled_sc(x_idx, table_lin)
    return _mlp_tc(pooled, W1, b1, W2, b2, W3, b3)


# speedup vs baseline: 2.0971x; 1.1535x over previous
import functools

import jax
import jax.numpy as jnp
from jax import lax
from jax.experimental import pallas as pl
from jax.experimental.pallas import tpu as pltpu
from jax.experimental.pallas import tpu_sc as plsc

B = 4096
L = 200
D = 32
VOCAB = 1000000
NC = 2
NS = 16
NW = NC * NS
SPW = B // NW
CHUNK = 100
NCH = L // CHUNK
NBUF = 4
VBLK = 8192
VSUB = 128


def _relayout_body(t_ref, o_ref):
    x = t_ref[...]
    y = x.T
    nc = VBLK // VSUB
    rr = VSUB // 4
    yw = jnp.concatenate(
        [y[c * VSUB:(c + 1) * VSUB, :] for c in range(nc)], axis=1)
    ir = lax.broadcasted_iota(jnp.int32, (rr, VSUB), 0)
    iv = lax.broadcasted_iota(jnp.int32, (rr, VSUB), 1)
    for g in range(4):
        qt = (iv == 4 * ir + g).astype(jnp.float32)
        z = jnp.dot(qt, yw, preferred_element_type=jnp.float32)
        for c in range(nc):
            o_ref[pl.ds(c * rr, rr), pl.ds(g * D, D)] = (
                z[:, c * D:(c + 1) * D])


def _table_to_lin(table_t):
    return pl.pallas_call(
        _relayout_body,
        grid=((VOCAB + VBLK - 1) // VBLK,),
        in_specs=[pl.BlockSpec((D, VBLK), lambda i: (0, i))],
        out_specs=pl.BlockSpec((VBLK // 4, 4 * D), lambda i: (i, 0)),
        out_shape=jax.ShapeDtypeStruct((VOCAB // 4, 4 * D), jnp.float32),
    )(table_t)


def _pool_body(x_hbm, table_hbm, out_hbm, idx_v, rows0, rows1, rows2, rows3,
               out_v, sem0, sem1, sem2, sem3):
    wid = lax.axis_index("s") * NC + lax.axis_index("c")
    base_row = wid * SPW * NCH

    pltpu.sync_copy(x_hbm.at[pl.ds(base_row, SPW * NCH)], idx_v)

    rows = (rows0, rows1, rows2, rows3)
    sems = (sem0, sem1, sem2, sem3)

    def issue(s, b):
        for c in range(NCH):
            pltpu.async_copy(
                table_hbm.at[idx_v.at[s * NCH + c]],
                rows[b].at[pl.ds(c * CHUNK, CHUNK)],
                sems[b],
            )

    def wait(b):
        for c in range(NCH):
            pltpu.make_async_copy(
                table_hbm.at[idx_v.at[c]],
                rows[b].at[pl.ds(c * CHUNK, CHUNK)],
                sems[b],
            ).wait()

    for b in range(NBUF):
        issue(b, b)

    zeros = jnp.zeros((16,), jnp.float32)
    neginf = jnp.full((16,), -jnp.inf, jnp.float32)
    inv_l = jnp.float32(1.0 / L)

    def outer(i, _):
        for b in range(NBUF):
            s = NBUF * i + b
            rb = rows[b]
            wait(b)

            @plsc.parallel_loop(
                0, L, step=2, unroll=4,
                carry=(zeros, zeros, neginf, neginf,
                       zeros, zeros, neginf, neginf))
            def red(r, carry):
                sa0, sa1, ma0, ma1, sb0, sb1, mb0, mb1 = carry
                va0 = rb[r, pl.ds(0, 16)]
                va1 = rb[r, pl.ds(16, 16)]
                vb0 = rb[r + 1, pl.ds(0, 16)]
                vb1 = rb[r + 1, pl.ds(16, 16)]
                return (sa0 + va0, sa1 + va1,
                        jnp.maximum(ma0, va0), jnp.maximum(ma1, va1),
                        sb0 + vb0, sb1 + vb1,
                        jnp.maximum(mb0, vb0), jnp.maximum(mb1, vb1))

            sa0, sa1, ma0, ma1, sb0, sb1, mb0, mb1 = red

            @pl.when(s + NBUF < SPW)
            def _():
                issue(s + NBUF, b)

            out_v[s, pl.ds(0, 16)] = (sa0 + sb0) * inv_l
            out_v[s, pl.ds(16, 16)] = (sa1 + sb1) * inv_l
            out_v[s, pl.ds(32, 16)] = jnp.maximum(ma0, mb0)
            out_v[s, pl.ds(48, 16)] = jnp.maximum(ma1, mb1)
        return 0

    lax.fori_loop(0, SPW // NBUF, outer, 0)

    pltpu.sync_copy(out_v, out_hbm.at[pl.ds(wid * SPW, SPW)])


def _pooled_sc(x_idx, table_lin):
    mesh = plsc.VectorSubcoreMesh(core_axis_name="c", subcore_axis_name="s")
    f = pl.kernel(
        _pool_body,
        out_type=jax.ShapeDtypeStruct((B, 2 * D), jnp.float32),
        mesh=mesh,
        scratch_types=[
            pltpu.VMEM((SPW * NCH, CHUNK), jnp.int32),
            pltpu.VMEM((L, D), jnp.float32),
            pltpu.VMEM((L, D), jnp.float32),
            pltpu.VMEM((L, D), jnp.float32),
            pltpu.VMEM((L, D), jnp.float32),
            pltpu.VMEM((SPW, 2 * D), jnp.float32),
            pltpu.SemaphoreType.DMA,
            pltpu.SemaphoreType.DMA,
            pltpu.SemaphoreType.DMA,
            pltpu.SemaphoreType.DMA,
        ],
        compiler_params=pltpu.CompilerParams(use_tc_tiling_on_sc=False),
    )
    return f(x_idx, table_lin)


def _mlp_body(p_ref, w1_ref, b1_ref, w2_ref, b2_ref, w3_ref, b3_ref, o_ref):
    h = jnp.dot(p_ref[...], w1_ref[...], preferred_element_type=jnp.float32)
    h = jnp.maximum(h + b1_ref[...], 0.0)
    h = jnp.dot(h, w2_ref[...], preferred_element_type=jnp.float32)
    h = jnp.maximum(h + b2_ref[...], 0.0)
    h = jnp.dot(h, w3_ref[...], preferred_element_type=jnp.float32)
    o_ref[...] = jax.nn.sigmoid(h + b3_ref[...])


def _mlp_tc(pooled, W1, b1, W2, b2, W3, b3):
    blk = 1024
    return pl.pallas_call(
        _mlp_body,
        grid=(B // blk,),
        in_specs=[
            pl.BlockSpec((blk, 2 * D), lambda i: (i, 0)),
            pl.BlockSpec((2 * D, 128), lambda i: (0, 0)),
            pl.BlockSpec((1, 128), lambda i: (0, 0)),
            pl.BlockSpec((128, 64), lambda i: (0, 0)),
            pl.BlockSpec((1, 64), lambda i: (0, 0)),
            pl.BlockSpec((64, 1), lambda i: (0, 0)),
            pl.BlockSpec((1, 1), lambda i: (0, 0)),
        ],
        out_specs=pl.BlockSpec((blk, 1), lambda i: (i, 0)),
        out_shape=jax.ShapeDtypeStruct((B, 1), jnp.float32),
    )(pooled, W1, b1.reshape(1, 128), W2, b2.reshape(1, 64),
      W3, b3.reshape(1, 1))


def kernel(x, table, W1, b1, W2, b2, W3, b3):
    x_idx = x.astype(jnp.int32).reshape(B * NCH, CHUNK)
    table_lin = _table_to_lin(table.T).reshape(VOCAB, D)
    pooled = _pooled_sc(x_idx, table_lin)
    return _mlp_tc(pooled, W1, b1, W2, b2, W3, b3)


# speedup vs baseline: 2.1376x; 1.0193x over previous
import functools

import jax
import jax.numpy as jnp
from jax import lax
from jax.experimental import pallas as pl
from jax.experimental.pallas import tpu as pltpu
from jax.experimental.pallas import tpu_sc as plsc

B = 4096
L = 200
D = 32
VOCAB = 1000000
NC = 2
NS = 16
NW = NC * NS
SPW = B // NW
CHUNK = 100
NCH = L // CHUNK
NBUF = 4
VBLK = 16384
VSUB = 128


def _relayout_body(t_ref, o_ref):
    x = t_ref[...]
    y = x.T
    nc = VBLK // VSUB
    rr = VSUB // 4
    yw = jnp.concatenate(
        [y[c * VSUB:(c + 1) * VSUB, :] for c in range(nc)], axis=1)
    ir = lax.broadcasted_iota(jnp.int32, (rr, VSUB), 0)
    iv = lax.broadcasted_iota(jnp.int32, (rr, VSUB), 1)
    for g in range(4):
        qt = (iv == 4 * ir + g).astype(jnp.float32)
        z = jnp.dot(qt, yw, preferred_element_type=jnp.float32)
        for c in range(nc):
            o_ref[pl.ds(c * rr, rr), pl.ds(g * D, D)] = (
                z[:, c * D:(c + 1) * D])


def _table_to_lin(table_t):
    return pl.pallas_call(
        _relayout_body,
        grid=((VOCAB + VBLK - 1) // VBLK,),
        in_specs=[pl.BlockSpec((D, VBLK), lambda i: (0, i))],
        out_specs=pl.BlockSpec((VBLK // 4, 4 * D), lambda i: (i, 0)),
        out_shape=jax.ShapeDtypeStruct((VOCAB // 4, 4 * D), jnp.float32),
    )(table_t)


def _pool_body(x_hbm, table_hbm, out_hbm, idx_v, rows0, rows1, rows2, rows3,
               out_v, sem0, sem1, sem2, sem3):
    wid = lax.axis_index("s") * NC + lax.axis_index("c")
    base_row = wid * SPW * NCH

    pltpu.sync_copy(x_hbm.at[pl.ds(base_row, SPW * NCH)], idx_v)

    rows = (rows0, rows1, rows2, rows3)
    sems = (sem0, sem1, sem2, sem3)

    def issue(s, b):
        for c in range(NCH):
            pltpu.async_copy(
                table_hbm.at[idx_v.at[s * NCH + c]],
                rows[b].at[pl.ds(c * CHUNK, CHUNK)],
                sems[b],
            )

    def wait(b):
        for c in range(NCH):
            pltpu.make_async_copy(
                table_hbm.at[idx_v.at[c]],
                rows[b].at[pl.ds(c * CHUNK, CHUNK)],
                sems[b],
            ).wait()

    for b in range(NBUF):
        issue(b, b)

    zeros = jnp.zeros((16,), jnp.float32)
    neginf = jnp.full((16,), -jnp.inf, jnp.float32)
    inv_l = jnp.float32(1.0 / L)

    def outer(i, _):
        for b in range(NBUF):
            s = NBUF * i + b
            rb = rows[b]
            wait(b)

            @plsc.parallel_loop(
                0, L, step=2, unroll=4,
                carry=(zeros, zeros, neginf, neginf,
                       zeros, zeros, neginf, neginf))
            def red(r, carry):
                sa0, sa1, ma0, ma1, sb0, sb1, mb0, mb1 = carry
                va0 = rb[r, pl.ds(0, 16)]
                va1 = rb[r, pl.ds(16, 16)]
                vb0 = rb[r + 1, pl.ds(0, 16)]
                vb1 = rb[r + 1, pl.ds(16, 16)]
                return (sa0 + va0, sa1 + va1,
                        jnp.maximum(ma0, va0), jnp.maximum(ma1, va1),
                        sb0 + vb0, sb1 + vb1,
                        jnp.maximum(mb0, vb0), jnp.maximum(mb1, vb1))

            sa0, sa1, ma0, ma1, sb0, sb1, mb0, mb1 = red

            @pl.when(s + NBUF < SPW)
            def _():
                issue(s + NBUF, b)

            out_v[s, pl.ds(0, 16)] = (sa0 + sb0) * inv_l
            out_v[s, pl.ds(16, 16)] = (sa1 + sb1) * inv_l
            out_v[s, pl.ds(32, 16)] = jnp.maximum(ma0, mb0)
            out_v[s, pl.ds(48, 16)] = jnp.maximum(ma1, mb1)
        return 0

    lax.fori_loop(0, SPW // NBUF, outer, 0)

    pltpu.sync_copy(out_v, out_hbm.at[pl.ds(wid * SPW, SPW)])


def _pooled_sc(x_idx, table_lin):
    mesh = plsc.VectorSubcoreMesh(core_axis_name="c", subcore_axis_name="s")
    f = pl.kernel(
        _pool_body,
        out_type=jax.ShapeDtypeStruct((B, 2 * D), jnp.float32),
        mesh=mesh,
        scratch_types=[
            pltpu.VMEM((SPW * NCH, CHUNK), jnp.int32),
            pltpu.VMEM((L, D), jnp.float32),
            pltpu.VMEM((L, D), jnp.float32),
            pltpu.VMEM((L, D), jnp.float32),
            pltpu.VMEM((L, D), jnp.float32),
            pltpu.VMEM((SPW, 2 * D), jnp.float32),
            pltpu.SemaphoreType.DMA,
            pltpu.SemaphoreType.DMA,
            pltpu.SemaphoreType.DMA,
            pltpu.SemaphoreType.DMA,
        ],
        compiler_params=pltpu.CompilerParams(use_tc_tiling_on_sc=False),
    )
    return f(x_idx, table_lin)


def _mlp_body(p_ref, w1_ref, b1_ref, w2_ref, b2_ref, w3_ref, b3_ref, o_ref):
    h = jnp.dot(p_ref[...], w1_ref[...], preferred_element_type=jnp.float32)
    h = jnp.maximum(h + b1_ref[...], 0.0)
    h = jnp.dot(h, w2_ref[...], preferred_element_type=jnp.float32)
    h = jnp.maximum(h + b2_ref[...], 0.0)
    h = jnp.dot(h, w3_ref[...], preferred_element_type=jnp.float32)
    o_ref[...] = jax.nn.sigmoid(h + b3_ref[...])


def _mlp_tc(pooled, W1, b1, W2, b2, W3, b3):
    blk = 1024
    return pl.pallas_call(
        _mlp_body,
        grid=(B // blk,),
        in_specs=[
            pl.BlockSpec((blk, 2 * D), lambda i: (i, 0)),
            pl.BlockSpec((2 * D, 128), lambda i: (0, 0)),
            pl.BlockSpec((1, 128), lambda i: (0, 0)),
            pl.BlockSpec((128, 64), lambda i: (0, 0)),
            pl.BlockSpec((1, 64), lambda i: (0, 0)),
            pl.BlockSpec((64, 1), lambda i: (0, 0)),
            pl.BlockSpec((1, 1), lambda i: (0, 0)),
        ],
        out_specs=pl.BlockSpec((blk, 1), lambda i: (i, 0)),
        out_shape=jax.ShapeDtypeStruct((B, 1), jnp.float32),
    )(pooled, W1, b1.reshape(1, 128), W2, b2.reshape(1, 64),
      W3, b3.reshape(1, 1))


def kernel(x, table, W1, b1, W2, b2, W3, b3):
    x_idx = x.astype(jnp.int32).reshape(B * NCH, CHUNK)
    table_lin = _table_to_lin(table.T).reshape(VOCAB, D)
    pooled = _pooled_sc(x_idx, table_lin)
    return _mlp_tc(pooled, W1, b1, W2, b2, W3, b3)
